# Initial kernel scaffold; baseline (speedup 1.0000x reference)
import jax
import jax.numpy as jnp
from jax.experimental import pallas as pl


def kernel(node_types, edge_index, task_finished, ag_node_indices, task_node_indices, W_emb, b_emb, gnn_W, gnn_b, W_bi, w_ag):
    raise NotImplementedError("write your pallas kernel here")


# speedup vs baseline: 5.7596x; 5.7596x over previous
import functools

import jax
import jax.numpy as jnp
from jax import lax
from jax.experimental import pallas as pl
from jax.experimental.pallas import tpu as pltpu
from jax.experimental.pallas import tpu_sc as plsc

N = 10000
D = 128
E = 320000
N_AG = 64
N_TASK = 256

NC = 2
NS = 16
NW = NC * NS
EPW = E // NW
C = 80
NCH = EPW // C
NPAD = 10240
RPT = NPAD // NS
RZ = 128
DPT = NPAD // NS


def _embed_body(t_ref, w_ref, b_ref, out_ref):
    t = t_ref[...]
    w0 = w_ref[0:1, :]
    w1 = w_ref[1:2, :]
    w2 = w_ref[2:3, :]
    out_ref[...] = jnp.where(t == 0, w0, jnp.where(t == 1, w1, w2)) + b_ref[...]


def _embed(t, w_emb, b2):
    r = 1000
    return pl.pallas_call(
        _embed_body,
        grid=(N // r,),
        in_specs=[
            pl.BlockSpec((r, 1), lambda i: (i, 0)),
            pl.BlockSpec((3, D), lambda i: (0, 0)),
            pl.BlockSpec((1, D), lambda i: (0, 0)),
        ],
        out_specs=pl.BlockSpec((r, D), lambda i: (i, 0)),
        out_shape=jax.ShapeDtypeStruct((N, D), jnp.float32),
    )(t, w_emb, b2)


def _sc_agg_build(with_deg):
    mesh = plsc.VectorSubcoreMesh(core_axis_name="c", subcore_axis_name="s",
                                  num_cores=NC, num_subcores=NS)
    out_type = [jax.ShapeDtypeStruct((NC, NPAD, D), jnp.float32)]
    scratch = [
        pltpu.VMEM_SHARED((NPAD, D), jnp.float32),
        pltpu.VMEM((C,), jnp.int32),
        pltpu.VMEM((C,), jnp.int32),
        pltpu.VMEM((C, D), jnp.float32),
        pltpu.VMEM((RZ, D), jnp.float32),
        pltpu.SemaphoreType.DMA,
    ]
    if with_deg:
        out_type.append(jax.ShapeDtypeStruct((NC, NPAD), jnp.float32))
        scratch += [
            pltpu.VMEM_SHARED((NPAD,), jnp.float32),
            pltpu.VMEM((C,), jnp.float32),
            pltpu.VMEM((DPT,), jnp.float32),
        ]

    def body(h_hbm, src_hbm, dst_hbm, *rest):
        if with_deg:
            (m_out, deg_out, m_sh, sidx, didx, rows, zbuf, sem,
             deg_sh, ones, dstage) = rest
        else:
            (m_out, m_sh, sidx, didx, rows, zbuf, sem) = rest
        cid = lax.axis_index("c")
        sid = lax.axis_index("s")
        wid = cid * NS + sid

        def zb(i, _):
            zbuf[i >> 3, pl.ds((i & 7) * 16, 16)] = jnp.zeros((16,), jnp.float32)
            return 0
        lax.fori_loop(0, RZ * (D // 16), zb, 0)
        for j in range(RPT // RZ):
            pltpu.sync_copy(zbuf, m_sh.at[pl.ds(sid * RPT + j * RZ, RZ)])
        if with_deg:
            def ob(i, _):
                ones[pl.ds(i * 16, 16)] = jnp.ones((16,), jnp.float32)
                return 0
            lax.fori_loop(0, C // 16, ob, 0)

            def db(i, _):
                dstage[pl.ds(i * 16, 16)] = jnp.zeros((16,), jnp.float32)
                return 0
            lax.fori_loop(0, DPT // 16, db, 0)
            pltpu.sync_copy(dstage, deg_sh.at[pl.ds(sid * DPT, DPT)])
        plsc.subcore_barrier()

        def step(it, _):
            base = wid * EPW + it * C
            pltpu.sync_copy(src_hbm.at[pl.ds(base, C)], sidx)
            pltpu.sync_copy(dst_hbm.at[pl.ds(base, C)], didx)
            pltpu.async_copy(h_hbm.at[sidx], rows, sem).wait()
            pltpu.sync_copy(rows, m_sh.at[didx], add=True)
            if with_deg:
                pltpu.sync_copy(ones, deg_sh.at[didx], add=True)
            return 0
        lax.fori_loop(0, NCH, step, 0)
        plsc.subcore_barrier()

        pltpu.sync_copy(m_sh.at[pl.ds(sid * RPT, RPT)],
                        m_out.at[cid, pl.ds(sid * RPT, RPT)])
        if with_deg:
            pltpu.sync_copy(deg_sh.at[pl.ds(sid * DPT, DPT)],
                            deg_out.at[cid, pl.ds(sid * DPT, DPT)])

    return pl.kernel(body, out_type=out_type, mesh=mesh, scratch_types=scratch)


@functools.lru_cache(maxsize=None)
def _sc_agg_get(with_deg):
    return _sc_agg_build(with_deg)


def _agg_deg(h, src, dst):
    return _sc_agg_get(True)(h, src, dst)


def _agg(h, src, dst):
    return _sc_agg_get(False)(h, src, dst)[0]


def _update_body(h_ref, m0_ref, m1_ref, d0_ref, d1_ref, w_ref, b_ref, out_ref):
    deg = jnp.maximum(d0_ref[...] + d1_ref[...], 1.0)
    m = (m0_ref[...] + m1_ref[...]) / deg
    acc = jnp.dot(m, w_ref[...], preferred_element_type=jnp.float32) + b_ref[...]
    out_ref[...] = h_ref[...] + jnp.maximum(acc, 0.0)


def _update(h, m0, m1, d0, d1, w, b2):
    r = 1000
    return pl.pallas_call(
        _update_body,
        grid=(N // r,),
        in_specs=[
            pl.BlockSpec((r, D), lambda i: (i, 0)),
            pl.BlockSpec((r, D), lambda i: (i, 0)),
            pl.BlockSpec((r, D), lambda i: (i, 0)),
            pl.BlockSpec((r, 1), lambda i: (i, 0)),
            pl.BlockSpec((r, 1), lambda i: (i, 0)),
            pl.BlockSpec((D, D), lambda i: (0, 0)),
            pl.BlockSpec((1, D), lambda i: (0, 0)),
        ],
        out_specs=pl.BlockSpec((r, D), lambda i: (i, 0)),
        out_shape=jax.ShapeDtypeStruct((N, D), jnp.float32),
    )(h, m0, m1, d0, d1, w, b2)


def _policy_body(h_ref, m0_ref, m1_ref, d0_ref, d1_ref, agi_ref, tski_ref,
                 w2_ref, b2_ref, wbi_ref, wag_ref, tf_ref, g1_ref, g2_ref,
                 outa_ref, outb_ref):
    f32 = jnp.float32
    deg = jnp.maximum(d0_ref[...] + d1_ref[...], 1.0)
    m = (m0_ref[...] + m1_ref[...]) / deg
    h2 = h_ref[...]

    oh_a = (lax.broadcasted_iota(jnp.int32, (N_AG, N), 1)
            == agi_ref[...]).astype(f32)
    oh_t = (lax.broadcasted_iota(jnp.int32, (N_TASK, N), 1)
            == tski_ref[...]).astype(f32)
    ag_h = jnp.dot(oh_a, h2, preferred_element_type=f32)
    ag_m = jnp.dot(oh_a, m, preferred_element_type=f32)
    t_h = jnp.dot(oh_t, h2, preferred_element_type=f32)
    t_m = jnp.dot(oh_t, m, preferred_element_type=f32)

    w2 = w2_ref[...]
    b2 = b2_ref[...]
    ag_nf = ag_h + jnp.maximum(
        jnp.dot(ag_m, w2, preferred_element_type=f32) + b2, 0.0)
    t_nf = t_h + jnp.maximum(
        jnp.dot(t_m, w2, preferred_element_type=f32) + b2, 0.0)

    s_a = jnp.dot(ag_nf, wbi_ref[...], preferred_element_type=f32)
    scores = lax.dot_general(s_a, t_nf, (((1,), (1,)), ((), ())),
                             preferred_element_type=f32)
    smax = jnp.max(scores, axis=1, keepdims=True)
    sexp = jnp.exp(scores - smax)
    jp0 = sexp / jnp.sum(sexp, axis=1, keepdims=True)
    jp0 = jnp.where(tf_ref[...] != 0, 0.0, jp0)

    av = lax.dot_general(wag_ref[...], ag_nf, (((1,), (1,)), ((), ())),
                         preferred_element_type=f32)
    amax = jnp.max(av, axis=1, keepdims=True)
    aexp = jnp.exp(av - amax)
    ap0 = aexp / jnp.sum(aexp, axis=1, keepdims=True)

    iota64 = lax.broadcasted_iota(jnp.int32, (1, N_AG), 1)
    iota256 = lax.broadcasted_iota(jnp.int32, (1, N_TASK), 1)
    big = jnp.int32(2 ** 30)
    eps = 1e-20

    def step(itr, carry):
        ap, jp, outa, outb = carry
        g1 = g1_ref[pl.ds(itr, 1), :]
        la = jnp.log(ap + eps) + g1
        lamax = jnp.max(la, axis=1, keepdims=True)
        aidx = jnp.min(jnp.where(la == lamax, iota64, big),
                       axis=1, keepdims=True)
        afirst = iota64 == aidx
        row = jnp.dot(afirst.astype(f32), jp, preferred_element_type=f32)
        g2 = g2_ref[pl.ds(itr, 1), :]
        lt = jnp.log(row + eps) + g2
        ltmax = jnp.max(lt, axis=1, keepdims=True)
        tidx = jnp.min(jnp.where(lt == ltmax, iota256, big),
                       axis=1, keepdims=True)
        tfirst = iota256 == tidx
        ap = jnp.where(afirst, 0.0, ap)
        jp = jnp.where(tfirst, 0.0, jp)
        sel = iota64 == itr
        outa = jnp.where(sel, jnp.broadcast_to(aidx, (1, N_AG)), outa)
        outb = jnp.where(sel, jnp.broadcast_to(tidx, (1, N_AG)), outb)
        return ap, jp, outa, outb

    init = (ap0, jp0,
            jnp.zeros((1, N_AG), jnp.int32), jnp.zeros((1, N_AG), jnp.int32))
    _, _, outa, outb = lax.fori_loop(0, N_AG, step, init)
    outa_ref[...] = outa
    outb_ref[...] = outb


def _policy(h2, m0, m1, d0, d1, agi, tski, w2, b2, wbi, wag, tf, g1, g2):
    full = lambda s: pl.BlockSpec(s, lambda: tuple(0 for _ in s))
    return pl.pallas_call(
        _policy_body,
        in_specs=[
            full((N, D)), full((N, D)), full((N, D)),
            full((N, 1)), full((N, 1)),
            full((N_AG, 1)), full((N_TASK, 1)),
            full((D, D)), full((1, D)), full((D, D)), full((1, D)),
            full((1, N_TASK)), full((N_AG, N_AG)), full((N_AG, N_TASK)),
        ],
        out_specs=[full((1, N_AG)), full((1, N_AG))],
        out_shape=[jax.ShapeDtypeStruct((1, N_AG), jnp.int32),
                   jax.ShapeDtypeStruct((1, N_AG), jnp.int32)],
    )(h2, m0, m1, d0, d1, agi, tski, w2, b2, wbi, wag, tf, g1, g2)


def kernel(node_types, edge_index, task_finished, ag_node_indices,
           task_node_indices, W_emb, b_emb, gnn_W, gnn_b, W_bi, w_ag):
    t = node_types.astype(jnp.int32).reshape(N, 1)
    src = edge_index[0].astype(jnp.int32)
    dst = edge_index[1].astype(jnp.int32)
    b2 = b_emb.reshape(1, D).astype(jnp.float32)

    nf = _embed(t, W_emb.astype(jnp.float32), b2)
    (m1p, degp) = _agg_deg(nf, src, dst)
    d0 = degp[0, :N].reshape(N, 1)
    d1 = degp[1, :N].reshape(N, 1)
    h1 = _update(nf, m1p[0, :N], m1p[1, :N], d0, d1,
                 gnn_W[0], gnn_b[0].reshape(1, D))
    m2p = _agg(h1, src, dst)
    h2 = _update(h1, m2p[0, :N], m2p[1, :N], d0, d1,
                 gnn_W[1], gnn_b[1].reshape(1, D))
    m3p = _agg(h2, src, dst)

    skey = jax.random.key(42)
    its = jnp.arange(N_AG)
    k1 = jax.vmap(lambda i: jax.random.fold_in(skey, 2 * i))(its)
    k2 = jax.vmap(lambda i: jax.random.fold_in(skey, 2 * i + 1))(its)
    g1 = jax.vmap(lambda k: jax.random.gumbel(k, (N_AG,), jnp.float32))(k1)
    g2 = jax.vmap(lambda k: jax.random.gumbel(k, (N_TASK,), jnp.float32))(k2)

    agi = ag_node_indices.astype(jnp.int32).reshape(N_AG, 1)
    tski = task_node_indices.astype(jnp.int32).reshape(N_TASK, 1)
    tf = task_finished.astype(jnp.int32).reshape(1, N_TASK)

    outa, outb = _policy(h2, m3p[0, :N], m3p[1, :N], d0, d1, agi, tski,
                         gnn_W[2], gnn_b[2].reshape(1, D), W_bi,
                         w_ag.reshape(1, D), tf, g1, g2)
    return outa.reshape(N_AG), outb.reshape(N_AG)


# speedup vs baseline: 11.5326x; 2.0023x over previous
import functools

import jax
import jax.numpy as jnp
from jax import lax
from jax.experimental import pallas as pl
from jax.experimental.pallas import tpu as pltpu
from jax.experimental.pallas import tpu_sc as plsc

N = 10000
D = 128
E = 320000
N_AG = 64
N_TASK = 256

NC = 2
NS = 16
NW = NC * NS
EPW = E // NW
C = 40
NCH = EPW // C
NPAD = 10240
RPT = NPAD // NS
RZ = 64
DPT = NPAD // NS


def _embed_body(t_ref, w_ref, b_ref, out_ref):
    t = t_ref[...]
    w0 = w_ref[0:1, :]
    w1 = w_ref[1:2, :]
    w2 = w_ref[2:3, :]
    out_ref[...] = jnp.where(t == 0, w0, jnp.where(t == 1, w1, w2)) + b_ref[...]


def _embed(t, w_emb, b2):
    r = 1000
    return pl.pallas_call(
        _embed_body,
        grid=(N // r,),
        in_specs=[
            pl.BlockSpec((r, 1), lambda i: (i, 0)),
            pl.BlockSpec((3, D), lambda i: (0, 0)),
            pl.BlockSpec((1, D), lambda i: (0, 0)),
        ],
        out_specs=pl.BlockSpec((r, D), lambda i: (i, 0)),
        out_shape=jax.ShapeDtypeStruct((N, D), jnp.float32),
    )(t, w_emb, b2)


NB = 5
NGRP = NCH // NB


def _sc_agg_build(with_deg):
    mesh = plsc.VectorSubcoreMesh(core_axis_name="c", subcore_axis_name="s",
                                  num_cores=NC, num_subcores=NS)
    out_type = [jax.ShapeDtypeStruct((NC, NPAD, D), jnp.float32)]
    scratch = [
        pltpu.VMEM_SHARED((NPAD, D), jnp.float32),
        pltpu.VMEM((RZ, D), jnp.float32),
    ]
    scratch += [pltpu.VMEM((2 * NB, C), jnp.int32) for _ in range(2)]
    scratch += [pltpu.VMEM((C, D), jnp.float32) for _ in range(NB)]
    scratch += [pltpu.SemaphoreType.DMA for _ in range(2 * NB + 2)]
    if with_deg:
        out_type.append(jax.ShapeDtypeStruct((NC, NPAD), jnp.float32))
        scratch += [
            pltpu.VMEM_SHARED((NPAD,), jnp.float32),
            pltpu.VMEM((C,), jnp.float32),
            pltpu.VMEM((DPT,), jnp.float32),
        ]

    def body(h_hbm, sd_hbm, *rest):
        if with_deg:
            (m_out, deg_out, m_sh, zbuf, *tail) = rest
            deg_sh, ones, dstage = tail[3 * NB + 4:]
        else:
            (m_out, m_sh, zbuf, *tail) = rest
        gbuf = tail[:2]
        rows = tail[2:2 + NB]
        gsem = tail[2 + NB:2 + 2 * NB]
        ssem = tail[2 + 2 * NB:2 + 3 * NB]
        glsem = tail[2 + 3 * NB:2 + 3 * NB + 2]
        cid = lax.axis_index("c")
        sid = lax.axis_index("s")
        wid = cid * NS + sid

        def zb(i, _):
            zbuf[i >> 3, pl.ds((i & 7) * 16, 16)] = jnp.zeros((16,), jnp.float32)
            return 0
        lax.fori_loop(0, RZ * (D // 16), zb, 0)
        for j in range(RPT // RZ):
            pltpu.sync_copy(zbuf, m_sh.at[pl.ds(sid * RPT + j * RZ, RZ)])
        if with_deg:
            def ob(i, _):
                ones[pl.ds(i * 16, 16)] = jnp.ones((16,), jnp.float32)
                return 0
            lax.fori_loop(0, C // 16, ob, 0)

            def db(i, _):
                dstage[pl.ds(i * 16, 16)] = jnp.zeros((16,), jnp.float32)
                return 0
            lax.fori_loop(0, DPT // 16, db, 0)
            pltpu.sync_copy(dstage, deg_sh.at[pl.ds(sid * DPT, DPT)])
        plsc.subcore_barrier()

        pltpu.async_copy(sd_hbm.at[wid, 0], gbuf[0], glsem[0])

        def group(k, p, _):
            gg = 2 * k + p
            pltpu.make_async_copy(sd_hbm.at[wid, 0], gbuf[p], glsem[p]).wait()
            gath = []
            for b in range(NB):
                def swait():
                    pltpu.make_async_copy(rows[b], m_sh.at[gbuf[p].at[NB + b]],
                                          ssem[b]).wait()
                if p == 0:
                    pl.when(k > 0)(swait)
                else:
                    swait()
                gath.append(pltpu.async_copy(h_hbm.at[gbuf[p].at[b]],
                                             rows[b], gsem[b]))
            def gload():
                pltpu.async_copy(sd_hbm.at[wid, gg + 1], gbuf[1 - p],
                                 glsem[1 - p])
            if p == 0:
                gload()
            else:
                pl.when(k < NGRP // 2 - 1)(gload)
            for b in range(NB):
                gath[b].wait()
                pltpu.async_copy(rows[b], m_sh.at[gbuf[p].at[NB + b]],
                                 ssem[b], add=True)
                if with_deg:
                    pltpu.sync_copy(ones, deg_sh.at[gbuf[p].at[NB + b]],
                                    add=True)
            return 0

        def pair(k, c):
            c = group(k, 0, c)
            c = group(k, 1, c)
            return c
        lax.fori_loop(0, NGRP // 2, pair, 0)
        for b in range(NB):
            pltpu.make_async_copy(rows[b], m_sh.at[gbuf[1].at[NB + b]],
                                  ssem[b]).wait()
        plsc.subcore_barrier()

        pltpu.sync_copy(m_sh.at[pl.ds(sid * RPT, RPT)],
                        m_out.at[cid, pl.ds(sid * RPT, RPT)])
        if with_deg:
            pltpu.sync_copy(deg_sh.at[pl.ds(sid * DPT, DPT)],
                            deg_out.at[cid, pl.ds(sid * DPT, DPT)])

    return pl.kernel(body, out_type=out_type, mesh=mesh, scratch_types=scratch)


@functools.lru_cache(maxsize=None)
def _sc_agg_get(with_deg):
    return _sc_agg_build(with_deg)


def _agg_deg(h, sd):
    return _sc_agg_get(True)(h, sd)


def _agg(h, sd):
    return _sc_agg_get(False)(h, sd)[0]


def _update_body(h_ref, m0_ref, m1_ref, d0_ref, d1_ref, w_ref, b_ref, out_ref):
    deg = jnp.maximum(d0_ref[...] + d1_ref[...], 1.0)
    m = (m0_ref[...] + m1_ref[...]) / deg
    acc = jnp.dot(m, w_ref[...], preferred_element_type=jnp.float32) + b_ref[...]
    out_ref[...] = h_ref[...] + jnp.maximum(acc, 0.0)


def _update(h, m0, m1, d0, d1, w, b2):
    r = 1000
    return pl.pallas_call(
        _update_body,
        grid=(N // r,),
        in_specs=[
            pl.BlockSpec((r, D), lambda i: (i, 0)),
            pl.BlockSpec((r, D), lambda i: (i, 0)),
            pl.BlockSpec((r, D), lambda i: (i, 0)),
            pl.BlockSpec((r, 1), lambda i: (i, 0)),
            pl.BlockSpec((r, 1), lambda i: (i, 0)),
            pl.BlockSpec((D, D), lambda i: (0, 0)),
            pl.BlockSpec((1, D), lambda i: (0, 0)),
        ],
        out_specs=pl.BlockSpec((r, D), lambda i: (i, 0)),
        out_shape=jax.ShapeDtypeStruct((N, D), jnp.float32),
    )(h, m0, m1, d0, d1, w, b2)


def _policy_body(h_ref, m0_ref, m1_ref, d0_ref, d1_ref, agi_ref, tski_ref,
                 w2_ref, b2_ref, wbi_ref, wag_ref, tf_ref, g1_ref, g2_ref,
                 outa_ref, outb_ref):
    f32 = jnp.float32
    deg = jnp.maximum(d0_ref[...] + d1_ref[...], 1.0)
    m = (m0_ref[...] + m1_ref[...]) / deg
    h2 = h_ref[...]

    oh_a = (lax.broadcasted_iota(jnp.int32, (N_AG, N), 1)
            == agi_ref[...]).astype(f32)
    oh_t = (lax.broadcasted_iota(jnp.int32, (N_TASK, N), 1)
            == tski_ref[...]).astype(f32)
    ag_h = jnp.dot(oh_a, h2, preferred_element_type=f32)
    ag_m = jnp.dot(oh_a, m, preferred_element_type=f32)
    t_h = jnp.dot(oh_t, h2, preferred_element_type=f32)
    t_m = jnp.dot(oh_t, m, preferred_element_type=f32)

    w2 = w2_ref[...]
    b2 = b2_ref[...]
    ag_nf = ag_h + jnp.maximum(
        jnp.dot(ag_m, w2, preferred_element_type=f32) + b2, 0.0)
    t_nf = t_h + jnp.maximum(
        jnp.dot(t_m, w2, preferred_element_type=f32) + b2, 0.0)

    s_a = jnp.dot(ag_nf, wbi_ref[...], preferred_element_type=f32)
    scores = lax.dot_general(s_a, t_nf, (((1,), (1,)), ((), ())),
                             preferred_element_type=f32)
    smax = jnp.max(scores, axis=1, keepdims=True)
    sexp = jnp.exp(scores - smax)
    jp0 = sexp / jnp.sum(sexp, axis=1, keepdims=True)
    jp0 = jnp.where(tf_ref[...] != 0, 0.0, jp0)

    av = lax.dot_general(wag_ref[...], ag_nf, (((1,), (1,)), ((), ())),
                         preferred_element_type=f32)
    amax = jnp.max(av, axis=1, keepdims=True)
    aexp = jnp.exp(av - amax)
    ap0 = aexp / jnp.sum(aexp, axis=1, keepdims=True)

    iota64 = lax.broadcasted_iota(jnp.int32, (1, N_AG), 1)
    iota256 = lax.broadcasted_iota(jnp.int32, (1, N_TASK), 1)
    big = jnp.int32(2 ** 30)
    eps = 1e-20

    def step(itr, carry):
        ap, jp, outa, outb = carry
        g1 = g1_ref[pl.ds(itr, 1), :]
        la = jnp.log(ap + eps) + g1
        lamax = jnp.max(la, axis=1, keepdims=True)
        aidx = jnp.min(jnp.where(la == lamax, iota64, big),
                       axis=1, keepdims=True)
        afirst = iota64 == aidx
        row = jnp.dot(afirst.astype(f32), jp, preferred_element_type=f32)
        g2 = g2_ref[pl.ds(itr, 1), :]
        lt = jnp.log(row + eps) + g2
        ltmax = jnp.max(lt, axis=1, keepdims=True)
        tidx = jnp.min(jnp.where(lt == ltmax, iota256, big),
                       axis=1, keepdims=True)
        tfirst = iota256 == tidx
        ap = jnp.where(afirst, 0.0, ap)
        jp = jnp.where(tfirst, 0.0, jp)
        sel = iota64 == itr
        outa = jnp.where(sel, jnp.broadcast_to(aidx, (1, N_AG)), outa)
        outb = jnp.where(sel, jnp.broadcast_to(tidx, (1, N_AG)), outb)
        return ap, jp, outa, outb

    init = (ap0, jp0,
            jnp.zeros((1, N_AG), jnp.int32), jnp.zeros((1, N_AG), jnp.int32))
    _, _, outa, outb = lax.fori_loop(0, N_AG, step, init)
    outa_ref[...] = outa
    outb_ref[...] = outb


def _policy(h2, m0, m1, d0, d1, agi, tski, w2, b2, wbi, wag, tf, g1, g2):
    full = lambda s: pl.BlockSpec(s, lambda: tuple(0 for _ in s))
    return pl.pallas_call(
        _policy_body,
        in_specs=[
            full((N, D)), full((N, D)), full((N, D)),
            full((N, 1)), full((N, 1)),
            full((N_AG, 1)), full((N_TASK, 1)),
            full((D, D)), full((1, D)), full((D, D)), full((1, D)),
            full((1, N_TASK)), full((N_AG, N_AG)), full((N_AG, N_TASK)),
        ],
        out_specs=[full((1, N_AG)), full((1, N_AG))],
        out_shape=[jax.ShapeDtypeStruct((1, N_AG), jnp.int32),
                   jax.ShapeDtypeStruct((1, N_AG), jnp.int32)],
    )(h2, m0, m1, d0, d1, agi, tski, w2, b2, wbi, wag, tf, g1, g2)


def kernel(node_types, edge_index, task_finished, ag_node_indices,
           task_node_indices, W_emb, b_emb, gnn_W, gnn_b, W_bi, w_ag):
    t = node_types.astype(jnp.int32).reshape(N, 1)
    sd = (edge_index.astype(jnp.int32)
          .reshape(2, NW, NGRP, NB, C)
          .transpose(1, 2, 0, 3, 4)
          .reshape(NW, NGRP, 2 * NB, C))
    b2 = b_emb.reshape(1, D).astype(jnp.float32)

    nf = _embed(t, W_emb.astype(jnp.float32), b2)
    (m1p, degp) = _agg_deg(nf, sd)
    d0 = degp[0, :N].reshape(N, 1)
    d1 = degp[1, :N].reshape(N, 1)
    h1 = _update(nf, m1p[0, :N], m1p[1, :N], d0, d1,
                 gnn_W[0], gnn_b[0].reshape(1, D))
    m2p = _agg(h1, sd)
    h2 = _update(h1, m2p[0, :N], m2p[1, :N], d0, d1,
                 gnn_W[1], gnn_b[1].reshape(1, D))
    m3p = _agg(h2, sd)

    skey = jax.random.key(42)
    its = jnp.arange(N_AG)
    k1 = jax.vmap(lambda i: jax.random.fold_in(skey, 2 * i))(its)
    k2 = jax.vmap(lambda i: jax.random.fold_in(skey, 2 * i + 1))(its)
    g1 = jax.vmap(lambda k: jax.random.gumbel(k, (N_AG,), jnp.float32))(k1)
    g2 = jax.vmap(lambda k: jax.random.gumbel(k, (N_TASK,), jnp.float32))(k2)

    agi = ag_node_indices.astype(jnp.int32).reshape(N_AG, 1)
    tski = task_node_indices.astype(jnp.int32).reshape(N_TASK, 1)
    tf = task_finished.astype(jnp.int32).reshape(1, N_TASK)

    outa, outb = _policy(h2, m3p[0, :N], m3p[1, :N], d0, d1, agi, tski,
                         gnn_W[2], gnn_b[2].reshape(1, D), W_bi,
                         w_ag.reshape(1, D), tf, g1, g2)
    return outa.reshape(N_AG), outb.reshape(N_AG)


# speedup vs baseline: 14.8261x; 1.2856x over previous
import functools

import jax
import jax.numpy as jnp
from jax import lax
from jax.experimental import pallas as pl
from jax.experimental.pallas import tpu as pltpu
from jax.experimental.pallas import tpu_sc as plsc

N = 10000
D = 128
E = 320000
N_AG = 64
N_TASK = 256

NC = 2
NS = 16
NW = NC * NS
EPW = E // NW
C = 40
NCH = EPW // C
NPAD = 10240
RPT = NPAD // NS
RZ = 64
DPT = NPAD // NS


def _embed_body(t_ref, w_ref, b_ref, out_ref):
    t = t_ref[...]
    w0 = w_ref[0:1, :]
    w1 = w_ref[1:2, :]
    w2 = w_ref[2:3, :]
    out_ref[...] = jnp.where(t == 0, w0, jnp.where(t == 1, w1, w2)) + b_ref[...]


def _embed(t, w_emb, b2):
    r = 1000
    return pl.pallas_call(
        _embed_body,
        grid=(N // r,),
        in_specs=[
            pl.BlockSpec((r, 1), lambda i: (i, 0)),
            pl.BlockSpec((3, D), lambda i: (0, 0)),
            pl.BlockSpec((1, D), lambda i: (0, 0)),
        ],
        out_specs=pl.BlockSpec((r, D), lambda i: (i, 0)),
        out_shape=jax.ShapeDtypeStruct((N, D), jnp.float32),
    )(t, w_emb, b2)


NB = 5
NGRP = NCH // NB

NPAD3 = NPAD * 3
CH = 80
NCHH = EPW // CH
NBH = 5
HPT = NPAD3 // NS


def _sc_mesh():
    return plsc.VectorSubcoreMesh(core_axis_name="c", subcore_axis_name="s",
                                  num_cores=NC, num_subcores=NS)


def _sc_agg_build():
    out_type = [jax.ShapeDtypeStruct((NC, NPAD, D), jnp.float32)]
    scratch = [
        pltpu.VMEM_SHARED((NPAD, D), jnp.float32),
        pltpu.VMEM((RZ, D), jnp.float32),
    ]
    scratch += [pltpu.VMEM((2 * NB, C), jnp.int32) for _ in range(2)]
    scratch += [pltpu.VMEM((C, D), jnp.float32) for _ in range(NB)]
    scratch += [pltpu.SemaphoreType.DMA for _ in range(2 * NB + 2)]

    def body(h_hbm, sd_hbm, m_out, m_sh, zbuf, *tail):
        gbuf = tail[:2]
        rows = tail[2:2 + NB]
        gsem = tail[2 + NB:2 + 2 * NB]
        ssem = tail[2 + 2 * NB:2 + 3 * NB]
        glsem = tail[2 + 3 * NB:2 + 3 * NB + 2]
        cid = lax.axis_index("c")
        sid = lax.axis_index("s")
        wid = cid * NS + sid

        def zb(i, _):
            zbuf[i >> 3, pl.ds((i & 7) * 16, 16)] = jnp.zeros((16,), jnp.float32)
            return 0
        lax.fori_loop(0, RZ * (D // 16), zb, 0)
        for j in range(RPT // RZ):
            pltpu.sync_copy(zbuf, m_sh.at[pl.ds(sid * RPT + j * RZ, RZ)])
        plsc.subcore_barrier()

        pltpu.async_copy(sd_hbm.at[wid, 0], gbuf[0], glsem[0])

        def group(k, p, _):
            gg = 2 * k + p
            pltpu.make_async_copy(sd_hbm.at[wid, 0], gbuf[p], glsem[p]).wait()
            gath = []
            for b in range(NB):
                def swait():
                    pltpu.make_async_copy(rows[b], m_sh.at[gbuf[p].at[NB + b]],
                                          ssem[b]).wait()
                if p == 0:
                    pl.when(k > 0)(swait)
                else:
                    swait()
                gath.append(pltpu.async_copy(h_hbm.at[gbuf[p].at[b]],
                                             rows[b], gsem[b]))
            def gload():
                pltpu.async_copy(sd_hbm.at[wid, gg + 1], gbuf[1 - p],
                                 glsem[1 - p])
            if p == 0:
                gload()
            else:
                pl.when(k < NGRP // 2 - 1)(gload)
            for b in range(NB):
                gath[b].wait()
                pltpu.async_copy(rows[b], m_sh.at[gbuf[p].at[NB + b]],
                                 ssem[b], add=True)
            return 0

        def pair(k, c):
            c = group(k, 0, c)
            c = group(k, 1, c)
            return c
        lax.fori_loop(0, NGRP // 2, pair, 0)
        for b in range(NB):
            pltpu.make_async_copy(rows[b], m_sh.at[gbuf[1].at[NB + b]],
                                  ssem[b]).wait()
        plsc.subcore_barrier()

        pltpu.sync_copy(m_sh.at[pl.ds(sid * RPT, RPT)],
                        m_out.at[cid, pl.ds(sid * RPT, RPT)])

    return pl.kernel(body, out_type=out_type, mesh=_sc_mesh(),
                     scratch_types=scratch)


def _sc_hist_build():
    out_type = [jax.ShapeDtypeStruct((NC, NPAD3), jnp.float32)]
    scratch = [
        pltpu.VMEM_SHARED((NPAD3,), jnp.float32),
        pltpu.VMEM((HPT,), jnp.float32),
        pltpu.VMEM((CH,), jnp.float32),
        pltpu.VMEM((NCHH, CH), jnp.int32),
        pltpu.VMEM((NCHH, CH), jnp.int32),
    ]
    scratch += [pltpu.VMEM((CH,), jnp.int32) for _ in range(NBH)]
    scratch += [pltpu.VMEM((CH,), jnp.int32) for _ in range(NBH)]
    scratch += [pltpu.SemaphoreType.DMA for _ in range(2 * NBH)]

    def body(nt_hbm, srch_hbm, dsth_hbm, cnt_out, cnt_sh, zstage, ones,
             sidx2, didx2, *tail):
        tsrc = tail[:NBH]
        fidx = tail[NBH:2 * NBH]
        gsem = tail[2 * NBH:3 * NBH]
        ssem = tail[3 * NBH:4 * NBH]
        cid = lax.axis_index("c")
        sid = lax.axis_index("s")
        wid = cid * NS + sid

        pltpu.sync_copy(srch_hbm.at[wid], sidx2)
        pltpu.sync_copy(dsth_hbm.at[wid], didx2)

        def zb(i, _):
            zstage[pl.ds(i * 16, 16)] = jnp.zeros((16,), jnp.float32)
            return 0
        lax.fori_loop(0, HPT // 16, zb, 0)

        def ob(i, _):
            ones[pl.ds(i * 16, 16)] = jnp.ones((16,), jnp.float32)
            return 0
        lax.fori_loop(0, CH // 16, ob, 0)
        pltpu.sync_copy(zstage, cnt_sh.at[pl.ds(sid * HPT, HPT)])
        plsc.subcore_barrier()

        for b in range(NBH):
            pltpu.async_copy(nt_hbm.at[sidx2.at[b]], tsrc[b], gsem[b])

        def outer(k, _):
            for b in range(NBH):
                it = k * NBH + b
                pltpu.make_async_copy(nt_hbm.at[sidx2.at[0]], tsrc[b],
                                      gsem[b]).wait()

                def swait():
                    pltpu.make_async_copy(ones, cnt_sh.at[fidx[b]],
                                          ssem[b]).wait()
                pl.when(k > 0)(swait)

                def fx(j, _):
                    sl = pl.ds(j * 16, 16)
                    fidx[b][sl] = didx2[it, sl] * 3 + tsrc[b][sl]
                    return 0
                lax.fori_loop(0, CH // 16, fx, 0)
                pltpu.async_copy(ones, cnt_sh.at[fidx[b]], ssem[b], add=True)

                @pl.when(it + NBH < NCHH)
                def _():
                    pltpu.async_copy(nt_hbm.at[sidx2.at[it + NBH]], tsrc[b],
                                     gsem[b])
            return 0
        lax.fori_loop(0, NCHH // NBH, outer, 0)
        for b in range(NBH):
            pltpu.make_async_copy(ones, cnt_sh.at[fidx[b]], ssem[b]).wait()
        plsc.subcore_barrier()

        pltpu.sync_copy(cnt_sh.at[pl.ds(sid * HPT, HPT)],
                        cnt_out.at[cid, pl.ds(sid * HPT, HPT)])

    return pl.kernel(body, out_type=out_type, mesh=_sc_mesh(),
                     scratch_types=scratch)


@functools.lru_cache(maxsize=None)
def _sc_agg_get():
    return _sc_agg_build()


@functools.lru_cache(maxsize=None)
def _sc_hist_get():
    return _sc_hist_build()


def _hist(nt, srch, dsth):
    return _sc_hist_get()(nt, srch, dsth)[0]


def _agg(h, sd):
    return _sc_agg_get()(h, sd)[0]


def _update1_body(nf_ref, c0_ref, c1_ref, wemb_ref, bemb_ref, w_ref, b_ref,
                  out_ref, deg_ref):
    f32 = jnp.float32
    cnts = c0_ref[0] + c1_ref[0]
    deg_raw = jnp.sum(cnts, axis=1, keepdims=True)
    msum = (jnp.dot(cnts, wemb_ref[...], preferred_element_type=f32)
            + deg_raw * bemb_ref[...])
    deg = jnp.maximum(deg_raw, 1.0)
    acc = jnp.dot(msum / deg, w_ref[...], preferred_element_type=f32) + b_ref[...]
    out_ref[...] = nf_ref[...] + jnp.maximum(acc, 0.0)
    deg_ref[...] = deg


def _update1(nf, cnt, w_emb, bemb2, w, b2):
    r = 1000
    return pl.pallas_call(
        _update1_body,
        grid=(N // r,),
        in_specs=[
            pl.BlockSpec((r, D), lambda i: (i, 0)),
            pl.BlockSpec((1, r, 3), lambda i: (0, i, 0)),
            pl.BlockSpec((1, r, 3), lambda i: (1, i, 0)),
            pl.BlockSpec((3, D), lambda i: (0, 0)),
            pl.BlockSpec((1, D), lambda i: (0, 0)),
            pl.BlockSpec((D, D), lambda i: (0, 0)),
            pl.BlockSpec((1, D), lambda i: (0, 0)),
        ],
        out_specs=[pl.BlockSpec((r, D), lambda i: (i, 0)),
                   pl.BlockSpec((r, 1), lambda i: (i, 0))],
        out_shape=[jax.ShapeDtypeStruct((N, D), jnp.float32),
                   jax.ShapeDtypeStruct((N, 1), jnp.float32)],
    )(nf, cnt, cnt, w_emb, bemb2, w, b2)


def _update_body(h_ref, m0_ref, m1_ref, deg_ref, w_ref, b_ref, out_ref):
    m = (m0_ref[0] + m1_ref[0]) / deg_ref[...]
    acc = jnp.dot(m, w_ref[...], preferred_element_type=jnp.float32) + b_ref[...]
    out_ref[...] = h_ref[...] + jnp.maximum(acc, 0.0)


def _update(h, m, deg, w, b2):
    r = 1000
    return pl.pallas_call(
        _update_body,
        grid=(N // r,),
        in_specs=[
            pl.BlockSpec((r, D), lambda i: (i, 0)),
            pl.BlockSpec((1, r, D), lambda i: (0, i, 0)),
            pl.BlockSpec((1, r, D), lambda i: (1, i, 0)),
            pl.BlockSpec((r, 1), lambda i: (i, 0)),
            pl.BlockSpec((D, D), lambda i: (0, 0)),
            pl.BlockSpec((1, D), lambda i: (0, 0)),
        ],
        out_specs=pl.BlockSpec((r, D), lambda i: (i, 0)),
        out_shape=jax.ShapeDtypeStruct((N, D), jnp.float32),
    )(h, m, m, deg, w, b2)


def _policy_body(h_ref, m0_ref, m1_ref, deg_ref, agi_ref, tski_ref,
                 w2_ref, b2_ref, wbi_ref, wag_ref, tf_ref, g1_ref, g2_ref,
                 outa_ref, outb_ref):
    f32 = jnp.float32
    m = (m0_ref[0] + m1_ref[0]) / deg_ref[...]
    h2 = h_ref[...]

    oh_a = (lax.broadcasted_iota(jnp.int32, (N_AG, N), 1)
            == agi_ref[...]).astype(f32)
    oh_t = (lax.broadcasted_iota(jnp.int32, (N_TASK, N), 1)
            == tski_ref[...]).astype(f32)
    ag_h = jnp.dot(oh_a, h2, preferred_element_type=f32)
    ag_m = jnp.dot(oh_a, m, preferred_element_type=f32)
    t_h = jnp.dot(oh_t, h2, preferred_element_type=f32)
    t_m = jnp.dot(oh_t, m, preferred_element_type=f32)

    w2 = w2_ref[...]
    b2 = b2_ref[...]
    ag_nf = ag_h + jnp.maximum(
        jnp.dot(ag_m, w2, preferred_element_type=f32) + b2, 0.0)
    t_nf = t_h + jnp.maximum(
        jnp.dot(t_m, w2, preferred_element_type=f32) + b2, 0.0)

    s_a = jnp.dot(ag_nf, wbi_ref[...], preferred_element_type=f32)
    scores = lax.dot_general(s_a, t_nf, (((1,), (1,)), ((), ())),
                             preferred_element_type=f32)
    smax = jnp.max(scores, axis=1, keepdims=True)
    sexp = jnp.exp(scores - smax)
    jp0 = sexp / jnp.sum(sexp, axis=1, keepdims=True)
    jp0 = jnp.where(tf_ref[...] != 0, 0.0, jp0)

    av = lax.dot_general(wag_ref[...], ag_nf, (((1,), (1,)), ((), ())),
                         preferred_element_type=f32)
    amax = jnp.max(av, axis=1, keepdims=True)
    aexp = jnp.exp(av - amax)
    ap0 = aexp / jnp.sum(aexp, axis=1, keepdims=True)

    iota64 = lax.broadcasted_iota(jnp.int32, (1, N_AG), 1)
    iota256 = lax.broadcasted_iota(jnp.int32, (1, N_TASK), 1)
    big = jnp.int32(2 ** 30)
    eps = 1e-20

    def step(itr, carry):
        ap, jp, outa, outb = carry
        g1 = g1_ref[pl.ds(itr, 1), :]
        la = jnp.log(ap + eps) + g1
        lamax = jnp.max(la, axis=1, keepdims=True)
        aidx = jnp.min(jnp.where(la == lamax, iota64, big),
                       axis=1, keepdims=True)
        afirst = iota64 == aidx
        row = jnp.dot(afirst.astype(f32), jp, preferred_element_type=f32)
        g2 = g2_ref[pl.ds(itr, 1), :]
        lt = jnp.log(row + eps) + g2
        ltmax = jnp.max(lt, axis=1, keepdims=True)
        tidx = jnp.min(jnp.where(lt == ltmax, iota256, big),
                       axis=1, keepdims=True)
        tfirst = iota256 == tidx
        ap = jnp.where(afirst, 0.0, ap)
        jp = jnp.where(tfirst, 0.0, jp)
        sel = iota64 == itr
        outa = jnp.where(sel, jnp.broadcast_to(aidx, (1, N_AG)), outa)
        outb = jnp.where(sel, jnp.broadcast_to(tidx, (1, N_AG)), outb)
        return ap, jp, outa, outb

    init = (ap0, jp0,
            jnp.zeros((1, N_AG), jnp.int32), jnp.zeros((1, N_AG), jnp.int32))
    _, _, outa, outb = lax.fori_loop(0, N_AG, step, init)
    outa_ref[...] = outa
    outb_ref[...] = outb


def _policy(h2, m, deg, agi, tski, w2, b2, wbi, wag, tf, g1, g2):
    full = lambda s: pl.BlockSpec(s, lambda i: tuple(0 for _ in s))
    return pl.pallas_call(
        _policy_body,
        grid=(1,),
        in_specs=[
            full((N, D)),
            pl.BlockSpec((1, N, D), lambda i: (0, 0, 0)),
            pl.BlockSpec((1, N, D), lambda i: (1, 0, 0)),
            full((N, 1)),
            full((N_AG, 1)), full((N_TASK, 1)),
            full((D, D)), full((1, D)), full((D, D)), full((1, D)),
            full((1, N_TASK)), full((N_AG, N_AG)), full((N_AG, N_TASK)),
        ],
        out_specs=[full((1, N_AG)), full((1, N_AG))],
        out_shape=[jax.ShapeDtypeStruct((1, N_AG), jnp.int32),
                   jax.ShapeDtypeStruct((1, N_AG), jnp.int32)],
    )(h2, m, m, deg, agi, tski, w2, b2, wbi, wag, tf, g1, g2)


def kernel(node_types, edge_index, task_finished, ag_node_indices,
           task_node_indices, W_emb, b_emb, gnn_W, gnn_b, W_bi, w_ag):
    t = node_types.astype(jnp.int32).reshape(N, 1)
    nt = node_types.astype(jnp.int32)
    e32 = edge_index.astype(jnp.int32)
    sd = (e32.reshape(2, NW, NGRP, NB, C)
          .transpose(1, 2, 0, 3, 4)
          .reshape(NW, NGRP, 2 * NB, C))
    srch = e32[0].reshape(NW, NCHH, CH)
    dsth = e32[1].reshape(NW, NCHH, CH)
    b2 = b_emb.reshape(1, D).astype(jnp.float32)

    nf = _embed(t, W_emb.astype(jnp.float32), b2)
    cnt = _hist(nt, srch, dsth).reshape(NC, NPAD, 3)
    h1, deg = _update1(nf, cnt, W_emb.astype(jnp.float32), b2,
                       gnn_W[0], gnn_b[0].reshape(1, D))
    m2p = _agg(h1, sd)
    h2 = _update(h1, m2p, deg, gnn_W[1], gnn_b[1].reshape(1, D))
    m3p = _agg(h2, sd)

    skey = jax.random.key(42)
    its = jnp.arange(N_AG)
    k1 = jax.vmap(lambda i: jax.random.fold_in(skey, 2 * i))(its)
    k2 = jax.vmap(lambda i: jax.random.fold_in(skey, 2 * i + 1))(its)
    g1 = jax.vmap(lambda k: jax.random.gumbel(k, (N_AG,), jnp.float32))(k1)
    g2 = jax.vmap(lambda k: jax.random.gumbel(k, (N_TASK,), jnp.float32))(k2)

    agi = ag_node_indices.astype(jnp.int32).reshape(N_AG, 1)
    tski = task_node_indices.astype(jnp.int32).reshape(N_TASK, 1)
    tf = task_finished.astype(jnp.int32).reshape(1, N_TASK)

    outa, outb = _policy(h2, m3p, deg, agi, tski,
                         gnn_W[2], gnn_b[2].reshape(1, D), W_bi,
                         w_ag.reshape(1, D), tf, g1, g2)
    return outa.reshape(N_AG), outb.reshape(N_AG)


# speedup vs baseline: 15.5417x; 1.0483x over previous
import functools

import jax
import jax.numpy as jnp
from jax import lax
from jax.experimental import pallas as pl
from jax.experimental.pallas import tpu as pltpu
from jax.experimental.pallas import tpu_sc as plsc

N = 10000
D = 128
E = 320000
N_AG = 64
N_TASK = 256

NC = 2
NS = 16
NW = NC * NS
EPW = E // NW
C = 40
NCH = EPW // C
NPAD = 10240
RPT = NPAD // NS
RZ = 64
DPT = NPAD // NS


NB = 5
NGRP = NCH // NB

NPAD3 = NPAD * 3
CH = 80
NCHH = EPW // CH
NBH = 5
HPT = NPAD3 // NS


def _sc_mesh():
    return plsc.VectorSubcoreMesh(core_axis_name="c", subcore_axis_name="s",
                                  num_cores=NC, num_subcores=NS)


NSEL = 512
SPT = NSEL // NS


def _sc_agg_build(gather):
    if gather:
        out_type = [jax.ShapeDtypeStruct((NC, NSEL, D), jnp.float32),
                    jax.ShapeDtypeStruct((NSEL, D), jnp.float32),
                    jax.ShapeDtypeStruct((NSEL,), jnp.float32)]
    else:
        out_type = [jax.ShapeDtypeStruct((NC, NPAD, D), jnp.float32)]
    scratch = [
        pltpu.VMEM_SHARED((NPAD, D), jnp.float32),
        pltpu.VMEM((RZ, D), jnp.float32),
    ]
    scratch += [pltpu.VMEM((2 * NB, C), jnp.int32) for _ in range(2)]
    scratch += [pltpu.VMEM((C, D), jnp.float32) for _ in range(NB)]
    scratch += [pltpu.SemaphoreType.DMA for _ in range(2 * NB + 2)]
    if gather:
        scratch += [pltpu.VMEM((SPT,), jnp.int32),
                    pltpu.VMEM((SPT, D), jnp.float32),
                    pltpu.VMEM((SPT,), jnp.float32)]

    def body(h_hbm, sd_hbm, *rest):
        if gather:
            (sel_hbm, deg_hbm, msel_out, hsel_out, dsel_out,
             m_sh, zbuf, *tail) = rest
            selbuf, grow, drow = tail[3 * NB + 4:]
        else:
            (m_out, m_sh, zbuf, *tail) = rest
        gbuf = tail[:2]
        rows = tail[2:2 + NB]
        gsem = tail[2 + NB:2 + 2 * NB]
        ssem = tail[2 + 2 * NB:2 + 3 * NB]
        glsem = tail[2 + 3 * NB:2 + 3 * NB + 2]
        cid = lax.axis_index("c")
        sid = lax.axis_index("s")
        wid = cid * NS + sid

        def zb(i, _):
            zbuf[i >> 3, pl.ds((i & 7) * 16, 16)] = jnp.zeros((16,), jnp.float32)
            return 0
        lax.fori_loop(0, RZ * (D // 16), zb, 0)
        for j in range(RPT // RZ):
            pltpu.sync_copy(zbuf, m_sh.at[pl.ds(sid * RPT + j * RZ, RZ)])
        plsc.subcore_barrier()

        pltpu.async_copy(sd_hbm.at[wid, 0], gbuf[0], glsem[0])

        def group(k, p, _):
            gg = 2 * k + p
            pltpu.make_async_copy(sd_hbm.at[wid, 0], gbuf[p], glsem[p]).wait()
            gath = []
            for b in range(NB):
                def swait():
                    pltpu.make_async_copy(rows[b], m_sh.at[gbuf[p].at[NB + b]],
                                          ssem[b]).wait()
                if p == 0:
                    pl.when(k > 0)(swait)
                else:
                    swait()
                gath.append(pltpu.async_copy(h_hbm.at[gbuf[p].at[b]],
                                             rows[b], gsem[b]))
            def gload():
                pltpu.async_copy(sd_hbm.at[wid, gg + 1], gbuf[1 - p],
                                 glsem[1 - p])
            if p == 0:
                gload()
            else:
                pl.when(k < NGRP // 2 - 1)(gload)
            for b in range(NB):
                gath[b].wait()
                pltpu.async_copy(rows[b], m_sh.at[gbuf[p].at[NB + b]],
                                 ssem[b], add=True)
            return 0

        def pair(k, c):
            c = group(k, 0, c)
            c = group(k, 1, c)
            return c
        lax.fori_loop(0, NGRP // 2, pair, 0)
        for b in range(NB):
            pltpu.make_async_copy(rows[b], m_sh.at[gbuf[1].at[NB + b]],
                                  ssem[b]).wait()
        plsc.subcore_barrier()

        if gather:
            pltpu.sync_copy(sel_hbm.at[pl.ds(sid * SPT, SPT)], selbuf)
            pltpu.async_copy(m_sh.at[selbuf], grow, gsem[0]).wait()
            pltpu.sync_copy(grow, msel_out.at[cid, pl.ds(sid * SPT, SPT)])

            @pl.when(cid == 0)
            def _():
                pltpu.async_copy(h_hbm.at[selbuf], grow, gsem[0]).wait()
                pltpu.sync_copy(grow, hsel_out.at[pl.ds(sid * SPT, SPT)])
                pltpu.async_copy(deg_hbm.at[selbuf], drow, gsem[0]).wait()
                pltpu.sync_copy(drow, dsel_out.at[pl.ds(sid * SPT, SPT)])
        else:
            pltpu.sync_copy(m_sh.at[pl.ds(sid * RPT, RPT)],
                            m_out.at[cid, pl.ds(sid * RPT, RPT)])

    return pl.kernel(body, out_type=out_type, mesh=_sc_mesh(),
                     scratch_types=scratch)


def _sc_hist_build():
    out_type = [jax.ShapeDtypeStruct((NC, NPAD3), jnp.float32)]
    scratch = [
        pltpu.VMEM_SHARED((NPAD3,), jnp.float32),
        pltpu.VMEM((HPT,), jnp.float32),
        pltpu.VMEM((CH,), jnp.float32),
        pltpu.VMEM((NCHH, CH), jnp.int32),
        pltpu.VMEM((NCHH, CH), jnp.int32),
    ]
    scratch += [pltpu.VMEM((CH,), jnp.int32) for _ in range(NBH)]
    scratch += [pltpu.VMEM((CH,), jnp.int32) for _ in range(NBH)]
    scratch += [pltpu.SemaphoreType.DMA for _ in range(2 * NBH)]

    def body(nt_hbm, srch_hbm, dsth_hbm, cnt_out, cnt_sh, zstage, ones,
             sidx2, didx2, *tail):
        tsrc = tail[:NBH]
        fidx = tail[NBH:2 * NBH]
        gsem = tail[2 * NBH:3 * NBH]
        ssem = tail[3 * NBH:4 * NBH]
        cid = lax.axis_index("c")
        sid = lax.axis_index("s")
        wid = cid * NS + sid

        pltpu.sync_copy(srch_hbm.at[wid], sidx2)
        pltpu.sync_copy(dsth_hbm.at[wid], didx2)

        def zb(i, _):
            zstage[pl.ds(i * 16, 16)] = jnp.zeros((16,), jnp.float32)
            return 0
        lax.fori_loop(0, HPT // 16, zb, 0)

        def ob(i, _):
            ones[pl.ds(i * 16, 16)] = jnp.ones((16,), jnp.float32)
            return 0
        lax.fori_loop(0, CH // 16, ob, 0)
        pltpu.sync_copy(zstage, cnt_sh.at[pl.ds(sid * HPT, HPT)])
        plsc.subcore_barrier()

        for b in range(NBH):
            pltpu.async_copy(nt_hbm.at[sidx2.at[b]], tsrc[b], gsem[b])

        def outer(k, _):
            for b in range(NBH):
                it = k * NBH + b
                pltpu.make_async_copy(nt_hbm.at[sidx2.at[0]], tsrc[b],
                                      gsem[b]).wait()

                def swait():
                    pltpu.make_async_copy(ones, cnt_sh.at[fidx[b]],
                                          ssem[b]).wait()
                pl.when(k > 0)(swait)

                def fx(j, _):
                    sl = pl.ds(j * 16, 16)
                    fidx[b][sl] = didx2[it, sl] * 3 + tsrc[b][sl]
                    return 0
                lax.fori_loop(0, CH // 16, fx, 0)
                pltpu.async_copy(ones, cnt_sh.at[fidx[b]], ssem[b], add=True)

                @pl.when(it + NBH < NCHH)
                def _():
                    pltpu.async_copy(nt_hbm.at[sidx2.at[it + NBH]], tsrc[b],
                                     gsem[b])
            return 0
        lax.fori_loop(0, NCHH // NBH, outer, 0)
        for b in range(NBH):
            pltpu.make_async_copy(ones, cnt_sh.at[fidx[b]], ssem[b]).wait()
        plsc.subcore_barrier()

        pltpu.sync_copy(cnt_sh.at[pl.ds(sid * HPT, HPT)],
                        cnt_out.at[cid, pl.ds(sid * HPT, HPT)])

    return pl.kernel(body, out_type=out_type, mesh=_sc_mesh(),
                     scratch_types=scratch)


@functools.lru_cache(maxsize=None)
def _sc_agg_get(gather=False):
    return _sc_agg_build(gather)


@functools.lru_cache(maxsize=None)
def _sc_hist_get():
    return _sc_hist_build()


def _hist(nt, srch, dsth):
    return _sc_hist_get()(nt, srch, dsth)[0]


def _agg(h, sd):
    return _sc_agg_get()(h, sd)[0]


def _agg_gather(h, sd, sel, deg_flat):
    return _sc_agg_get(True)(h, sd, sel, deg_flat)


def _update1_body(t_ref, c0_ref, c1_ref, wemb_ref, bemb_ref, w_ref, b_ref,
                  out_ref, deg_ref):
    f32 = jnp.float32
    t = t_ref[...]
    nf = jnp.where(t == 0, wemb_ref[0:1, :],
                   jnp.where(t == 1, wemb_ref[1:2, :],
                             wemb_ref[2:3, :])) + bemb_ref[...]
    cnts = c0_ref[0] + c1_ref[0]
    deg_raw = jnp.sum(cnts, axis=1, keepdims=True)
    msum = (jnp.dot(cnts, wemb_ref[...], preferred_element_type=f32)
            + deg_raw * bemb_ref[...])
    deg = jnp.maximum(deg_raw, 1.0)
    acc = jnp.dot(msum / deg, w_ref[...], preferred_element_type=f32) + b_ref[...]
    out_ref[...] = nf + jnp.maximum(acc, 0.0)
    deg_ref[...] = deg


def _update1(t, cnt, w_emb, bemb2, w, b2):
    r = 1000
    return pl.pallas_call(
        _update1_body,
        grid=(N // r,),
        in_specs=[
            pl.BlockSpec((r, 1), lambda i: (i, 0)),
            pl.BlockSpec((1, r, 3), lambda i: (0, i, 0)),
            pl.BlockSpec((1, r, 3), lambda i: (1, i, 0)),
            pl.BlockSpec((3, D), lambda i: (0, 0)),
            pl.BlockSpec((1, D), lambda i: (0, 0)),
            pl.BlockSpec((D, D), lambda i: (0, 0)),
            pl.BlockSpec((1, D), lambda i: (0, 0)),
        ],
        out_specs=[pl.BlockSpec((r, D), lambda i: (i, 0)),
                   pl.BlockSpec((r, 1), lambda i: (i, 0))],
        out_shape=[jax.ShapeDtypeStruct((N, D), jnp.float32),
                   jax.ShapeDtypeStruct((N, 1), jnp.float32)],
    )(t, cnt, cnt, w_emb, bemb2, w, b2)


def _update_body(h_ref, m0_ref, m1_ref, deg_ref, w_ref, b_ref, out_ref):
    m = (m0_ref[0] + m1_ref[0]) / deg_ref[...]
    acc = jnp.dot(m, w_ref[...], preferred_element_type=jnp.float32) + b_ref[...]
    out_ref[...] = h_ref[...] + jnp.maximum(acc, 0.0)


def _update(h, m, deg, w, b2):
    r = 1000
    return pl.pallas_call(
        _update_body,
        grid=(N // r,),
        in_specs=[
            pl.BlockSpec((r, D), lambda i: (i, 0)),
            pl.BlockSpec((1, r, D), lambda i: (0, i, 0)),
            pl.BlockSpec((1, r, D), lambda i: (1, i, 0)),
            pl.BlockSpec((r, 1), lambda i: (i, 0)),
            pl.BlockSpec((D, D), lambda i: (0, 0)),
            pl.BlockSpec((1, D), lambda i: (0, 0)),
        ],
        out_specs=pl.BlockSpec((r, D), lambda i: (i, 0)),
        out_shape=jax.ShapeDtypeStruct((N, D), jnp.float32),
    )(h, m, m, deg, w, b2)


def _policy_body(hs_ref, ms0_ref, ms1_ref, ds_ref,
                 w2_ref, b2_ref, wbi_ref, wag_ref, tf_ref, g1_ref, g2_ref,
                 outa_ref, outb_ref):
    f32 = jnp.float32
    m = (ms0_ref[0] + ms1_ref[0]) / ds_ref[...]
    nf3 = hs_ref[...] + jnp.maximum(
        jnp.dot(m, w2_ref[...], preferred_element_type=f32) + b2_ref[...], 0.0)
    ag_nf = nf3[0:N_AG]
    t_nf = nf3[N_AG:N_AG + N_TASK]

    s_a = jnp.dot(ag_nf, wbi_ref[...], preferred_element_type=f32)
    scores = lax.dot_general(s_a, t_nf, (((1,), (1,)), ((), ())),
                             preferred_element_type=f32)
    smax = jnp.max(scores, axis=1, keepdims=True)
    sexp = jnp.exp(scores - smax)
    jp0 = sexp / jnp.sum(sexp, axis=1, keepdims=True)
    jp0 = jnp.where(tf_ref[...] != 0, 0.0, jp0)

    av = lax.dot_general(wag_ref[...], ag_nf, (((1,), (1,)), ((), ())),
                         preferred_element_type=f32)
    amax = jnp.max(av, axis=1, keepdims=True)
    aexp = jnp.exp(av - amax)
    ap0 = aexp / jnp.sum(aexp, axis=1, keepdims=True)

    iota64 = lax.broadcasted_iota(jnp.int32, (1, N_AG), 1)
    iota64c = lax.broadcasted_iota(jnp.int32, (N_AG, 1), 0)
    iota256 = lax.broadcasted_iota(jnp.int32, (1, N_TASK), 1)
    big = jnp.int32(2 ** 30)
    eps = 1e-20

    la0 = jnp.log(ap0 + eps)
    lj0 = jnp.log(jp0 + eps)
    lzero = jnp.log(jnp.zeros((1, 1), f32) + eps)

    def step(itr, carry):
        lap, ljp, outa, outb = carry
        la = lap + g1_ref[pl.ds(itr, 1), :]
        lamax = jnp.max(la, axis=1, keepdims=True)
        aidx = jnp.min(jnp.where(la == lamax, iota64, big),
                       axis=1, keepdims=True)
        afirst = iota64 == aidx
        row = jnp.sum(jnp.where(iota64c == aidx, ljp, 0.0),
                      axis=0, keepdims=True)
        lt = row + g2_ref[pl.ds(itr, 1), :]
        ltmax = jnp.max(lt, axis=1, keepdims=True)
        tidx = jnp.min(jnp.where(lt == ltmax, iota256, big),
                       axis=1, keepdims=True)
        tfirst = iota256 == tidx
        lap = jnp.where(afirst, lzero, lap)
        ljp = jnp.where(tfirst, lzero, ljp)
        sel = iota64 == itr
        outa = jnp.where(sel, jnp.broadcast_to(aidx, (1, N_AG)), outa)
        outb = jnp.where(sel, jnp.broadcast_to(tidx, (1, N_AG)), outb)
        return lap, ljp, outa, outb

    init = (la0, lj0,
            jnp.zeros((1, N_AG), jnp.int32), jnp.zeros((1, N_AG), jnp.int32))
    _, _, outa, outb = lax.fori_loop(0, N_AG, step, init)
    outa_ref[...] = outa
    outb_ref[...] = outb


def _policy(hsel, msel, dsel, w2, b2, wbi, wag, tf, g1, g2):
    full = lambda s: pl.BlockSpec(s, lambda i: tuple(0 for _ in s))
    return pl.pallas_call(
        _policy_body,
        grid=(1,),
        in_specs=[
            full((NSEL, D)),
            pl.BlockSpec((1, NSEL, D), lambda i: (0, 0, 0)),
            pl.BlockSpec((1, NSEL, D), lambda i: (1, 0, 0)),
            full((NSEL, 1)),
            full((D, D)), full((1, D)), full((D, D)), full((1, D)),
            full((1, N_TASK)), full((N_AG, N_AG)), full((N_AG, N_TASK)),
        ],
        out_specs=[full((1, N_AG)), full((1, N_AG))],
        out_shape=[jax.ShapeDtypeStruct((1, N_AG), jnp.int32),
                   jax.ShapeDtypeStruct((1, N_AG), jnp.int32)],
    )(hsel, msel, msel, dsel, w2, b2, wbi, wag, tf, g1, g2)


def kernel(node_types, edge_index, task_finished, ag_node_indices,
           task_node_indices, W_emb, b_emb, gnn_W, gnn_b, W_bi, w_ag):
    t = node_types.astype(jnp.int32).reshape(N, 1)
    nt = node_types.astype(jnp.int32)
    e32 = edge_index.astype(jnp.int32)
    sd = (e32.reshape(2, NW, NGRP, NB, C)
          .transpose(1, 2, 0, 3, 4)
          .reshape(NW, NGRP, 2 * NB, C))
    srch = e32[0].reshape(NW, NCHH, CH)
    dsth = e32[1].reshape(NW, NCHH, CH)
    b2 = b_emb.reshape(1, D).astype(jnp.float32)

    cnt = _hist(nt, srch, dsth).reshape(NC, NPAD, 3)
    h1, deg = _update1(t, cnt, W_emb.astype(jnp.float32), b2,
                       gnn_W[0], gnn_b[0].reshape(1, D))
    m2p = _agg(h1, sd)
    h2 = _update(h1, m2p, deg, gnn_W[1], gnn_b[1].reshape(1, D))
    sel = jnp.concatenate([
        ag_node_indices.astype(jnp.int32),
        task_node_indices.astype(jnp.int32),
        jnp.zeros((NSEL - N_AG - N_TASK,), jnp.int32)])
    msel, hsel, dsel = _agg_gather(h2, sd, sel, deg.reshape(N))

    skey = jax.random.key(42)
    its = jnp.arange(N_AG)
    k1 = jax.vmap(lambda i: jax.random.fold_in(skey, 2 * i))(its)
    k2 = jax.vmap(lambda i: jax.random.fold_in(skey, 2 * i + 1))(its)
    g1 = jax.vmap(lambda k: jax.random.gumbel(k, (N_AG,), jnp.float32))(k1)
    g2 = jax.vmap(lambda k: jax.random.gumbel(k, (N_TASK,), jnp.float32))(k2)

    tf = task_finished.astype(jnp.int32).reshape(1, N_TASK)

    outa, outb = _policy(hsel, msel, dsel.reshape(NSEL, 1),
                         gnn_W[2], gnn_b[2].reshape(1, D), W_bi,
                         w_ag.reshape(1, D), tf, g1, g2)
    return outa.reshape(N_AG), outb.reshape(N_AG)
